# baseline (speedup 1.0000x reference)
import functools

import jax
import jax.numpy as jnp
from jax import lax
from jax.experimental import pallas as pl
from jax.experimental.pallas import tpu as pltpu
from jax.experimental.pallas import tpu_sc as plsc

_MAX_IDX = 64
_L = 16
_PBLK = 12


def _make_sc_pool(b, n, d, t):
    info = plsc.get_sparse_core_info()
    nw = info.num_cores * info.num_subcores
    spw = (b * n) // nw
    ngrp = spw // _L
    nvec = d // _L
    mesh = plsc.VectorSubcoreMesh(core_axis_name="c", subcore_axis_name="s")

    @functools.partial(
        pl.kernel,
        mesh=mesh,
        out_type=jax.ShapeDtypeStruct((b * n * d,), jnp.float32),
        scratch_types=[
            pltpu.VMEM((2 * spw,), jnp.int32),
            pltpu.VMEM(((_MAX_IDX + 1) * d,), jnp.float32),
            pltpu.VMEM((spw * d,), jnp.float32),
            pltpu.SemaphoreType.DMA,
            pltpu.SemaphoreType.DMA,
            [pltpu.SemaphoreType.DMA] * ngrp,
        ],
        compiler_params=pltpu.CompilerParams(needs_layout_passes=False),
    )
    def pool(seq_hbm, spans_hbm, out_hbm,
             se_v, p_v, out_v, sem_x, sem_p, sems_o):
        wid = lax.axis_index("s") * info.num_cores + lax.axis_index("c")
        base = wid * spw
        bi = base // n

        cp_x = pltpu.async_copy(
            seq_hbm.at[pl.ds(bi * t * d, _MAX_IDX * d)],
            p_v.at[pl.ds(d, _MAX_IDX * d)], sem_x)
        cp_p = pltpu.async_copy(
            spans_hbm.at[pl.ds(2 * base, 2 * spw)], se_v, sem_p)

        lane = lax.iota(jnp.int32, _L)
        zeros = jnp.zeros((_L,), jnp.int32)
        fzeros = jnp.zeros((_L,), jnp.float32)

        cp_p.wait()
        groups = []
        for g in range(ngrp):
            rows = lane + (g * _L)
            s16 = plsc.load_gather(se_v, [rows * 2])
            e16 = plsc.load_gather(se_v, [rows * 2 + 1])
            inv16 = 1.0 / (e16 - s16 + 1).astype(jnp.float32)
            groups.append((s16 * d, (e16 + 1) * d, inv16, rows * d))

        for c in range(nvec):
            p_v[pl.ds(c * _L, _L)] = fzeros
        cp_x.wait()
        for blk in range(0, nvec, _PBLK):
            nb = min(_PBLK, nvec - blk)

            def prefix_row(i, carry, blk=blk, nb=nb):
                off = i * d + blk * _L
                new = []
                for c in range(nb):
                    sl = pl.ds(off + c * _L, _L)
                    acc = carry[c] + p_v[sl]
                    p_v[sl] = acc
                    new.append(acc)
                return tuple(new)

            lax.fori_loop(1, _MAX_IDX + 1, prefix_row, (fzeros,) * nb)

        outs = []
        for g in range(ngrp):
            s16, e16, inv16, rows = groups[g]

            def col_body(c, carry, s16=s16, e16=e16, inv16=inv16, rows=rows):
                blk = (c >> 4) + lane
                blk = jnp.where(blk >= d // _L, blk - d // _L, blk)
                colv = (blk << 4) + ((lane + c) & (_L - 1))
                pe = plsc.load_gather(p_v, [e16 + colv])
                ps = plsc.load_gather(p_v, [s16 + colv])
                plsc.store_scatter(out_v, [rows + colv], (pe - ps) * inv16)
                return carry

            lax.fori_loop(0, d, col_body, 0, unroll=8)
            outs.append(pltpu.async_copy(
                out_v.at[pl.ds(g * _L * d, _L * d)],
                out_hbm.at[pl.ds((base + g * _L) * d, _L * d)],
                sems_o[g]))

        for cp in outs:
            cp.wait()

    return pool


def kernel(sequence_tensor, span_indices):
    b, t, d = sequence_tensor.shape
    n = span_indices.shape[1]
    seq_flat = sequence_tensor.reshape(b * t * d)
    spans_flat = span_indices.reshape(b * n * 2).astype(jnp.int32)
    pool = _make_sc_pool(b, n, d, t)
    out = pool(seq_flat, spans_flat)
    return out.reshape(b, n, d)


# speedup vs baseline: 1.0131x; 1.0131x over previous
import functools

import jax
import jax.numpy as jnp
from jax import lax
from jax.experimental import pallas as pl
from jax.experimental.pallas import tpu as pltpu
from jax.experimental.pallas import tpu_sc as plsc

_MAX_IDX = 64
_L = 16
_PBLK = 12


def _make_sc_pool(b, n, d, t):
    info = plsc.get_sparse_core_info()
    nw = info.num_cores * info.num_subcores
    spw = (b * n) // nw
    ngrp = spw // _L
    nvec = d // _L
    mesh = plsc.VectorSubcoreMesh(core_axis_name="c", subcore_axis_name="s")

    @functools.partial(
        pl.kernel,
        mesh=mesh,
        out_type=jax.ShapeDtypeStruct((b * n * d,), jnp.float32),
        scratch_types=[
            pltpu.VMEM((2 * spw,), jnp.int32),
            pltpu.VMEM((spw + _L,), jnp.int32),
            pltpu.VMEM((spw + _L,), jnp.int32),
            pltpu.VMEM((spw + _L,), jnp.float32),
            pltpu.VMEM(((_MAX_IDX + 1) * d,), jnp.float32),
            pltpu.VMEM((spw * d,), jnp.float32),
            pltpu.SemaphoreType.DMA,
            pltpu.SemaphoreType.DMA,
            [pltpu.SemaphoreType.DMA] * (spw // _L),
        ],
        compiler_params=pltpu.CompilerParams(needs_layout_passes=False),
    )
    def pool(seq_hbm, spans_hbm, out_hbm,
             se_v, off_s, off_e, inv_v, p_v, out_v, sem_x, sem_p, sems_o):
        wid = lax.axis_index("s") * info.num_cores + lax.axis_index("c")
        base = wid * spw
        bi = base // n

        cp_x = pltpu.async_copy(
            seq_hbm.at[pl.ds(bi * t * d, _MAX_IDX * d)],
            p_v.at[pl.ds(d, _MAX_IDX * d)], sem_x)
        cp_p = pltpu.async_copy(
            spans_hbm.at[pl.ds(2 * base, 2 * spw)], se_v, sem_p)

        lane = lax.iota(jnp.int32, _L)
        fzeros = jnp.zeros((_L,), jnp.float32)

        cp_p.wait()
        for g in range(ngrp):
            sl = pl.ds(g * _L, _L)
            rows = lane + (g * _L)
            s16 = plsc.load_gather(se_v, [rows * 2])
            e16 = plsc.load_gather(se_v, [rows * 2 + 1])
            off_s[sl] = s16 * d
            off_e[sl] = (e16 + 1) * d
            inv_v[sl] = 1.0 / (e16 - s16 + 1).astype(jnp.float32)

        for c in range(nvec):
            p_v[pl.ds(c * _L, _L)] = fzeros
        cp_x.wait()
        for blk in range(0, nvec, _PBLK):
            nb = min(_PBLK, nvec - blk)

            def prefix_row(i, carry, blk=blk, nb=nb):
                off = i * d + blk * _L
                new = []
                for c in range(nb):
                    sl = pl.ds(off + c * _L, _L)
                    acc = carry[c] + p_v[sl]
                    p_v[sl] = acc
                    new.append(acc)
                return tuple(new)

            lax.fori_loop(1, _MAX_IDX + 1, prefix_row, (fzeros,) * nb)

        outs = []
        for g in range(ngrp):
            def span_body(j, carry, goff=g * _L):
                rs = off_s[pl.ds(goff + j, _L)][0]
                re = off_e[pl.ds(goff + j, _L)][0]
                inv = inv_v[pl.ds(goff + j, _L)][0]
                obase = (goff + j) * d
                for c in range(nvec):
                    co = c * _L
                    out_v[pl.ds(obase + co, _L)] = (
                        p_v[pl.ds(re + co, _L)] - p_v[pl.ds(rs + co, _L)]
                    ) * inv
                return carry

            lax.fori_loop(0, _L, span_body, 0)
            outs.append(pltpu.async_copy(
                out_v.at[pl.ds(g * _L * d, _L * d)],
                out_hbm.at[pl.ds((base + g * _L) * d, _L * d)],
                sems_o[g]))

        for cp in outs:
            cp.wait()

    return pool


def kernel(sequence_tensor, span_indices):
    b, t, d = sequence_tensor.shape
    n = span_indices.shape[1]
    seq_flat = sequence_tensor.reshape(b * t * d)
    spans_flat = span_indices.reshape(b * n * 2).astype(jnp.int32)
    pool = _make_sc_pool(b, n, d, t)
    out = pool(seq_flat, spans_flat)
    return out.reshape(b, n, d)


# speedup vs baseline: 2.1078x; 2.0805x over previous
import functools

import numpy as np
import jax
import jax.numpy as jnp
from jax import lax
from jax.experimental import pallas as pl
from jax.experimental.pallas import tpu as pltpu
from jax.experimental.pallas import tpu_sc as plsc

_MAX_IDX = 64
_L = 16
_RCP_W = 8 * _L
_CHUNK = 16

_RCP_TABLE = np.broadcast_to(
    (1.0 / np.arange(1, _MAX_IDX + 1, dtype=np.float32))[:, None],
    (_MAX_IDX, _RCP_W),
).copy()


def _prefix_kernel(x_ref, p_ref):
    b = x_ref.shape[0]
    d = x_ref.shape[-1]
    x = x_ref[...].reshape(b * _MAX_IDX, d)
    pr = _MAX_IDX + 1
    rows = lax.broadcasted_iota(jnp.int32, (b * pr, b * _MAX_IDX), 0)
    cols = lax.broadcasted_iota(jnp.int32, (b * pr, b * _MAX_IDX), 1)
    rb = rows // pr
    cb = cols // _MAX_IDX
    tri = ((rb == cb) & (cols - cb * _MAX_IDX < rows - rb * pr))
    p_ref[...] = jax.lax.dot_general(
        tri.astype(jnp.float32), x, (((1,), (0,)), ((), ())),
        preferred_element_type=jnp.float32,
        precision=lax.Precision.HIGHEST,
    )


def _make_sc_extract(total_spans, d, spans_per_batch):
    info = plsc.get_sparse_core_info()
    nw = info.num_cores * info.num_subcores
    spw = total_spans // nw
    nch = spw // _CHUNK
    mesh = plsc.VectorSubcoreMesh(core_axis_name="c", subcore_axis_name="s")

    @functools.partial(
        pl.kernel,
        mesh=mesh,
        out_type=jax.ShapeDtypeStruct((total_spans, d), jnp.float32),
        scratch_types=[
            pltpu.VMEM((spw,), jnp.int32),
            pltpu.VMEM((spw,), jnp.int32),
            pltpu.VMEM((spw,), jnp.int32),
            pltpu.VMEM((spw,), jnp.int32),
            pltpu.VMEM((spw,), jnp.int32),
            pltpu.VMEM((spw, _RCP_W), jnp.float32),
            pltpu.VMEM((spw, d), jnp.float32),
            pltpu.VMEM((spw, d), jnp.float32),
            pltpu.SemaphoreType.DMA,
            [pltpu.SemaphoreType.DMA] * nch,
            [pltpu.SemaphoreType.DMA] * nch,
            [pltpu.SemaphoreType.DMA] * nch,
        ],
    )
    def extract(p_hbm, starts_hbm, ends_hbm, rcp_hbm, out_hbm,
                s_v, e_v, idx_s, idx_e, idx_w, inv_rows, rows_s, rows_e,
                sem_w, sems_s, sems_e, sems_o):
        wid = lax.axis_index("s") * info.num_cores + lax.axis_index("c")
        base = wid * spw
        rowoff = (base // spans_per_batch) * (_MAX_IDX + 1)

        pltpu.sync_copy(starts_hbm.at[pl.ds(base, spw)], s_v)
        pltpu.sync_copy(ends_hbm.at[pl.ds(base, spw)], e_v)

        for g in range(spw // _L):
            sl = pl.ds(g * _L, _L)
            s16 = s_v[sl]
            e16 = e_v[sl]
            idx_s[sl] = s16 + rowoff
            idx_e[sl] = e16 + (rowoff + 1)
            idx_w[sl] = e16 - s16

        cp_w = pltpu.async_copy(rcp_hbm.at[idx_w], inv_rows, sem_w)
        cps = []
        for k in range(nch):
            ck = pl.ds(k * _CHUNK, _CHUNK)
            cps.append((
                pltpu.async_copy(p_hbm.at[idx_e.at[ck]], rows_e.at[ck],
                                 sems_e[k]),
                pltpu.async_copy(p_hbm.at[idx_s.at[ck]], rows_s.at[ck],
                                 sems_s[k]),
            ))
        cp_w.wait()

        outs = []
        for k in range(nch):
            cp_e, cp_s = cps[k]
            cp_e.wait()
            cp_s.wait()

            @plsc.parallel_loop(0, _CHUNK, unroll=2)
            def scale_row(j, goff=k * _CHUNK):
                row = goff + j
                inv = inv_rows[row, pl.ds(0, _L)]
                for c in range(d // _L):
                    sl = pl.ds(c * _L, _L)
                    rows_e[row, sl] = (rows_e[row, sl] - rows_s[row, sl]) * inv
            ck = pl.ds(k * _CHUNK, _CHUNK)
            outs.append(pltpu.async_copy(
                rows_e.at[ck], out_hbm.at[pl.ds(base + k * _CHUNK, _CHUNK)],
                sems_o[k]))

        for cp in outs:
            cp.wait()

    return extract


def kernel(sequence_tensor, span_indices):
    b, _, d = sequence_tensor.shape
    n = span_indices.shape[1]

    p_flat = pl.pallas_call(
        _prefix_kernel,
        grid=(1,),
        in_specs=[pl.BlockSpec((b, _MAX_IDX, d), lambda i: (0, 0, 0))],
        out_specs=pl.BlockSpec((b * (_MAX_IDX + 1), d), lambda i: (0, 0)),
        out_shape=jax.ShapeDtypeStruct((b * (_MAX_IDX + 1), d), jnp.float32),
    )(sequence_tensor)

    starts = span_indices[..., 0].reshape(-1).astype(jnp.int32)
    ends = span_indices[..., 1].reshape(-1).astype(jnp.int32)

    extract = _make_sc_extract(b * n, d, n)
    out = extract(p_flat, starts, ends, _RCP_TABLE)
    return out.reshape(b, n, d)


# speedup vs baseline: 2.2304x; 1.0582x over previous
import functools

import numpy as np
import jax
import jax.numpy as jnp
from jax import lax
from jax.experimental import pallas as pl
from jax.experimental.pallas import tpu as pltpu
from jax.experimental.pallas import tpu_sc as plsc

_MAX_IDX = 64
_L = 16
_RCP_W = 8 * _L
_CHUNK = 16

_RCP_TABLE = np.broadcast_to(
    (1.0 / np.arange(1, _MAX_IDX + 1, dtype=np.float32))[:, None],
    (_MAX_IDX, _RCP_W),
).copy()


def _prefix_kernel(x_ref, p_ref):
    b = x_ref.shape[0]
    d = x_ref.shape[-1]
    x = x_ref[...].reshape(b * _MAX_IDX, d)
    pr = _MAX_IDX + 1
    rows = lax.broadcasted_iota(jnp.int32, (b * pr, b * _MAX_IDX), 0)
    cols = lax.broadcasted_iota(jnp.int32, (b * pr, b * _MAX_IDX), 1)
    rb = rows // pr
    cb = cols // _MAX_IDX
    tri = ((rb == cb) & (cols - cb * _MAX_IDX < rows - rb * pr))
    p_ref[...] = jax.lax.dot_general(
        tri.astype(jnp.float32), x, (((1,), (0,)), ((), ())),
        preferred_element_type=jnp.float32,
        precision=lax.Precision.HIGHEST,
    )


def _make_sc_extract(total_spans, d, spans_per_batch):
    info = plsc.get_sparse_core_info()
    nw = info.num_cores * info.num_subcores
    spw = total_spans // nw
    nch = spw // _CHUNK
    mesh = plsc.VectorSubcoreMesh(core_axis_name="c", subcore_axis_name="s")

    @functools.partial(
        pl.kernel,
        mesh=mesh,
        out_type=jax.ShapeDtypeStruct((total_spans, d), jnp.float32),
        scratch_types=[
            pltpu.VMEM((spw,), jnp.int32),
            pltpu.VMEM((spw,), jnp.int32),
            pltpu.VMEM((spw,), jnp.int32),
            pltpu.VMEM((spw,), jnp.int32),
            pltpu.VMEM((spw,), jnp.int32),
            pltpu.VMEM((spw, _RCP_W), jnp.float32),
            pltpu.VMEM((spw, d), jnp.float32),
            pltpu.VMEM((spw, d), jnp.float32),
            pltpu.SemaphoreType.DMA,
            [pltpu.SemaphoreType.DMA] * nch,
            [pltpu.SemaphoreType.DMA] * nch,
            [pltpu.SemaphoreType.DMA] * nch,
        ],
    )
    def extract(p_hbm, starts_hbm, ends_hbm, rcp_hbm, out_hbm,
                s_v, e_v, idx_s, idx_e, idx_w, inv_rows, rows_s, rows_e,
                sem_w, sems_s, sems_e, sems_o):
        wid = lax.axis_index("s") * info.num_cores + lax.axis_index("c")
        base = wid * spw
        rowoff = (base // spans_per_batch) * (_MAX_IDX + 1)

        pltpu.sync_copy(starts_hbm.at[pl.ds(base, spw)], s_v)
        pltpu.sync_copy(ends_hbm.at[pl.ds(base, spw)], e_v)

        for g in range(spw // _L):
            sl = pl.ds(g * _L, _L)
            s16 = s_v[sl]
            e16 = e_v[sl]
            idx_s[sl] = s16 + rowoff
            idx_e[sl] = e16 + (rowoff + 1)
            idx_w[sl] = e16 - s16

        cp_w = pltpu.async_copy(rcp_hbm.at[idx_w], inv_rows, sem_w)
        cps = []
        for k in range(nch):
            ck = pl.ds(k * _CHUNK, _CHUNK)
            cps.append((
                pltpu.async_copy(p_hbm.at[idx_e.at[ck]], rows_e.at[ck],
                                 sems_e[k]),
                pltpu.async_copy(p_hbm.at[idx_s.at[ck]], rows_s.at[ck],
                                 sems_s[k]),
            ))
        cp_w.wait()

        outs = []
        for k in range(nch):
            cp_e, cp_s = cps[k]
            cp_e.wait()
            cp_s.wait()

            def scale_row(j, carry, goff=k * _CHUNK):
                row = goff + j
                inv = inv_rows[row, pl.ds(0, _L)]
                for c in range(d // _L):
                    sl = pl.ds(c * _L, _L)
                    rows_e[row, sl] = (rows_e[row, sl] - rows_s[row, sl]) * inv
                return carry

            lax.fori_loop(0, _CHUNK, scale_row, 0)
            ck = pl.ds(k * _CHUNK, _CHUNK)
            outs.append(pltpu.async_copy(
                rows_e.at[ck], out_hbm.at[pl.ds(base + k * _CHUNK, _CHUNK)],
                sems_o[k]))

        for cp in outs:
            cp.wait()

    return extract


def kernel(sequence_tensor, span_indices):
    b, _, d = sequence_tensor.shape
    n = span_indices.shape[1]

    p_flat = pl.pallas_call(
        _prefix_kernel,
        grid=(1,),
        in_specs=[pl.BlockSpec((b, _MAX_IDX, d), lambda i: (0, 0, 0))],
        out_specs=pl.BlockSpec((b * (_MAX_IDX + 1), d), lambda i: (0, 0)),
        out_shape=jax.ShapeDtypeStruct((b * (_MAX_IDX + 1), d), jnp.float32),
    )(sequence_tensor)

    starts = span_indices[..., 0].reshape(-1).astype(jnp.int32)
    ends = span_indices[..., 1].reshape(-1).astype(jnp.int32)

    extract = _make_sc_extract(b * n, d, n)
    out = extract(p_flat, starts, ends, _RCP_TABLE)
    return out.reshape(b, n, d)
